# baseline (speedup 1.0000x reference)
import jax
import jax.numpy as jnp
from jax.experimental import pallas as pl
from jax.experimental.pallas import tpu as pltpu

_D_MODEL = 1024
_D_FFN = 4096
_TOP_K = 256
_BLK = 256


def _ffn_kernel(x_ref, wg_ref, wu_ref, wd_ref, o_ref):
    x = x_ref[...]
    g = jnp.dot(x, wg_ref[...], preferred_element_type=jnp.float32)
    u = jnp.dot(x.astype(jnp.bfloat16), wu_ref[...],
                preferred_element_type=jnp.float32).astype(jnp.bfloat16)

    bits = jax.lax.bitcast_convert_type(g, jnp.int32)
    key = bits ^ ((bits >> 31) & jnp.int32(0x7FFFFFFF))

    cnt_pos = jnp.sum(key >= 0, axis=1, keepdims=True, dtype=jnp.int32)
    t = jnp.where(cnt_pos >= _TOP_K, jnp.int32(0), jnp.int32(-(2**31)))
    for b in range(30, -1, -1):
        cand = t | jnp.int32(1 << b)
        cnt = jnp.sum(key >= cand, axis=1, keepdims=True, dtype=jnp.int32)
        t = jnp.where(cnt >= _TOP_K, cand, t)
    mask = key >= t

    silu_bf = jnp.where(mask, g * jax.nn.sigmoid(g), 0.0).astype(jnp.bfloat16)
    z = silu_bf * u
    o_ref[...] = jnp.dot(z, wd_ref[...], preferred_element_type=jnp.float32)


def kernel(x, w_gate, w_up, w_down):
    orig_shape = x.shape
    x2 = x.reshape(-1, _D_MODEL)
    n = x2.shape[0]
    wu = w_up.astype(jnp.bfloat16)
    wd = w_down.astype(jnp.bfloat16)
    out = pl.pallas_call(
        _ffn_kernel,
        grid=(n // _BLK,),
        in_specs=[
            pl.BlockSpec((_BLK, _D_MODEL), lambda i: (i, 0)),
            pl.BlockSpec((_D_MODEL, _D_FFN), lambda i: (0, 0)),
            pl.BlockSpec((_D_MODEL, _D_FFN), lambda i: (0, 0)),
            pl.BlockSpec((_D_FFN, _D_MODEL), lambda i: (0, 0)),
        ],
        out_specs=pl.BlockSpec((_BLK, _D_MODEL), lambda i: (i, 0)),
        out_shape=jax.ShapeDtypeStruct((n, _D_MODEL), jnp.float32),
        compiler_params=pltpu.CompilerParams(
            dimension_semantics=("arbitrary",),
            vmem_limit_bytes=64 * 1024 * 1024,
        ),
    )(x2, w_gate, wu, wd)
    return out.reshape(orig_shape)


# speedup vs baseline: 1.0193x; 1.0193x over previous
import jax
import jax.numpy as jnp
from jax.experimental import pallas as pl
from jax.experimental.pallas import tpu as pltpu

_D_MODEL = 1024
_D_FFN = 4096
_TOP_K = 256
_BLK = 256


def _ffn_kernel(x_ref, wg_ref, wu_ref, wd_ref, o_ref):
    x = x_ref[...]
    g = jnp.dot(x, wg_ref[...], preferred_element_type=jnp.float32)
    u = jnp.dot(x.astype(jnp.bfloat16), wu_ref[...],
                preferred_element_type=jnp.float32)

    bits = jax.lax.bitcast_convert_type(g, jnp.int32)
    key = bits ^ ((bits >> 31) & jnp.int32(0x7FFFFFFF))

    cnt_pos = jnp.sum(key >= 0, axis=1, keepdims=True, dtype=jnp.int32)
    t = jnp.where(cnt_pos >= _TOP_K, jnp.int32(0), jnp.int32(-(2**31)))
    for b in range(30, -1, -1):
        cand = t | jnp.int32(1 << b)
        cnt = jnp.sum(key >= cand, axis=1, keepdims=True, dtype=jnp.int32)
        t = jnp.where(cnt >= _TOP_K, cand, t)
    mask = key >= t

    z = jnp.where(mask, g * jax.nn.sigmoid(g) * u, 0.0)
    o_ref[...] = jnp.dot(z.astype(jnp.bfloat16), wd_ref[...],
                         preferred_element_type=jnp.float32)


def kernel(x, w_gate, w_up, w_down):
    orig_shape = x.shape
    x2 = x.reshape(-1, _D_MODEL)
    n = x2.shape[0]
    wu = w_up.astype(jnp.bfloat16)
    wd = w_down.astype(jnp.bfloat16)
    out = pl.pallas_call(
        _ffn_kernel,
        grid=(n // _BLK,),
        in_specs=[
            pl.BlockSpec((_BLK, _D_MODEL), lambda i: (i, 0)),
            pl.BlockSpec((_D_MODEL, _D_FFN), lambda i: (0, 0)),
            pl.BlockSpec((_D_MODEL, _D_FFN), lambda i: (0, 0)),
            pl.BlockSpec((_D_FFN, _D_MODEL), lambda i: (0, 0)),
        ],
        out_specs=pl.BlockSpec((_BLK, _D_MODEL), lambda i: (i, 0)),
        out_shape=jax.ShapeDtypeStruct((n, _D_MODEL), jnp.float32),
        compiler_params=pltpu.CompilerParams(
            dimension_semantics=("arbitrary",),
        ),
    )(x2, w_gate, wu, wd)
    return out.reshape(orig_shape)
